# baseline (speedup 1.0000x reference)
import jax
import jax.numpy as jnp
from jax.experimental import pallas as pl
from jax.experimental.pallas import tpu as pltpu


_ROWS, _COLS = 16384, 128
_NCHUNKS = 4
_CHUNK = (_ROWS - 8) // _NCHUNKS


def _dma_kernel(x_hbm, o_hbm, scr, sem_big, sem_s):
    bigs = []
    for c in range(_NCHUNKS):
        base = 8 + c * _CHUNK
        cp = pltpu.make_async_copy(
            x_hbm.at[pl.ds(base, _CHUNK), :],
            o_hbm.at[pl.ds(base, _CHUNK), :],
            sem_big,
        )
        cp.start()
        bigs.append(cp)

    sm_in = pltpu.make_async_copy(x_hbm.at[pl.ds(0, 8), :], scr, sem_s)
    sm_in.start()
    sm_in.wait()
    rows = jax.lax.broadcasted_iota(jnp.int32, (8, _COLS), 0)
    cols = jax.lax.broadcasted_iota(jnp.int32, (8, _COLS), 1)
    scr[...] = jnp.where((rows == 0) & (cols == 0), jnp.float32(3.0), scr[...])
    sm_out = pltpu.make_async_copy(scr, o_hbm.at[pl.ds(0, 8), :], sem_s)
    sm_out.start()
    sm_out.wait()
    for cp in bigs:
        cp.wait()


def kernel(x):
    return pl.pallas_call(
        _dma_kernel,
        in_specs=[pl.BlockSpec(memory_space=pl.ANY)],
        out_specs=pl.BlockSpec(memory_space=pl.ANY),
        out_shape=jax.ShapeDtypeStruct((_ROWS, _COLS), x.dtype),
        scratch_shapes=[
            pltpu.VMEM((8, _COLS), jnp.float32),
            pltpu.SemaphoreType.DMA,
            pltpu.SemaphoreType.DMA,
        ],
    )(x)


# speedup vs baseline: 10.0923x; 10.0923x over previous
import functools

import jax
import jax.numpy as jnp
from jax import lax
from jax.experimental import pallas as pl
from jax.experimental.pallas import tpu as pltpu
from jax.experimental.pallas import tpu_sc as plsc


_ROWS, _COLS = 16384, 128
_NW = 32
_RPW = _ROWS // _NW
_CHUNK = _RPW // 2


def _sc_copy(x_hbm, o_hbm, buf0, buf1, sem0, sem1):
    nc = lax.axis_size("c")
    wid = lax.axis_index("s") * nc + lax.axis_index("c")
    base = wid * _RPW

    in0 = pltpu.make_async_copy(
        x_hbm.at[pl.ds(base, _CHUNK), :], buf0, sem0)
    in1 = pltpu.make_async_copy(
        x_hbm.at[pl.ds(base + _CHUNK, _CHUNK), :], buf1, sem1)
    in0.start()
    in1.start()

    in0.wait()

    @pl.when(wid == 0)
    def _():
        lane = lax.iota(jnp.int32, 16)
        head = buf0[0, pl.ds(0, 16)]
        buf0[0, pl.ds(0, 16)] = jnp.where(lane == 0, jnp.float32(3.0), head)

    out0 = pltpu.make_async_copy(
        buf0, o_hbm.at[pl.ds(base, _CHUNK), :], sem0)
    out0.start()

    in1.wait()
    out1 = pltpu.make_async_copy(
        buf1, o_hbm.at[pl.ds(base + _CHUNK, _CHUNK), :], sem1)
    out1.start()

    out0.wait()
    out1.wait()


def kernel(x):
    mesh = plsc.VectorSubcoreMesh(core_axis_name="c", subcore_axis_name="s")
    run = functools.partial(
        pl.kernel,
        mesh=mesh,
        out_type=jax.ShapeDtypeStruct((_ROWS, _COLS), jnp.float32),
        scratch_types=[
            pltpu.VMEM((_CHUNK, _COLS), jnp.float32),
            pltpu.VMEM((_CHUNK, _COLS), jnp.float32),
            pltpu.SemaphoreType.DMA,
            pltpu.SemaphoreType.DMA,
        ],
    )(_sc_copy)
    return run(x)


# speedup vs baseline: 19.9137x; 1.9732x over previous
import jax
import jax.numpy as jnp
from jax.experimental import pallas as pl


_ROWS, _COLS = 16384, 128
_BLOCK_ROWS = 1024
_GRID = _ROWS // _BLOCK_ROWS


def _copy_set_kernel(x_ref, o_ref):
    o_ref[...] = x_ref[...]

    @pl.when(pl.program_id(0) == 0)
    def _():
        head = x_ref[pl.ds(0, 8), :]
        rows = jax.lax.broadcasted_iota(jnp.int32, (8, _COLS), 0)
        cols = jax.lax.broadcasted_iota(jnp.int32, (8, _COLS), 1)
        hit = (rows == 0) & (cols == 0)
        o_ref[pl.ds(0, 8), :] = jnp.where(hit, jnp.float32(3.0), head)


def kernel(x):
    return pl.pallas_call(
        _copy_set_kernel,
        grid=(_GRID,),
        in_specs=[pl.BlockSpec((_BLOCK_ROWS, _COLS), lambda i: (i, 0))],
        out_specs=pl.BlockSpec((_BLOCK_ROWS, _COLS), lambda i: (i, 0)),
        out_shape=jax.ShapeDtypeStruct((_ROWS, _COLS), x.dtype),
    )(x)


# speedup vs baseline: 35.1266x; 1.7639x over previous
import jax
import jax.numpy as jnp
from jax.experimental import pallas as pl


_ROWS, _COLS = 16384, 128
_BLOCK_ROWS = 4096
_GRID = _ROWS // _BLOCK_ROWS


def _copy_set_kernel(x_ref, o_ref):
    o_ref[...] = x_ref[...]

    @pl.when(pl.program_id(0) == 0)
    def _():
        head = x_ref[pl.ds(0, 8), :]
        rows = jax.lax.broadcasted_iota(jnp.int32, (8, _COLS), 0)
        cols = jax.lax.broadcasted_iota(jnp.int32, (8, _COLS), 1)
        hit = (rows == 0) & (cols == 0)
        o_ref[pl.ds(0, 8), :] = jnp.where(hit, jnp.float32(3.0), head)


def kernel(x):
    return pl.pallas_call(
        _copy_set_kernel,
        grid=(_GRID,),
        in_specs=[pl.BlockSpec((_BLOCK_ROWS, _COLS), lambda i: (i, 0))],
        out_specs=pl.BlockSpec((_BLOCK_ROWS, _COLS), lambda i: (i, 0)),
        out_shape=jax.ShapeDtypeStruct((_ROWS, _COLS), x.dtype),
    )(x)


# speedup vs baseline: 42.9581x; 1.2230x over previous
import jax
import jax.numpy as jnp
from jax.experimental import pallas as pl


_ROWS, _COLS = 16384, 128
_BLOCK_ROWS = 8192
_GRID = _ROWS // _BLOCK_ROWS


def _copy_set_kernel(x_ref, o_ref):
    o_ref[...] = x_ref[...]

    @pl.when(pl.program_id(0) == 0)
    def _():
        head = x_ref[pl.ds(0, 8), :]
        rows = jax.lax.broadcasted_iota(jnp.int32, (8, _COLS), 0)
        cols = jax.lax.broadcasted_iota(jnp.int32, (8, _COLS), 1)
        hit = (rows == 0) & (cols == 0)
        o_ref[pl.ds(0, 8), :] = jnp.where(hit, jnp.float32(3.0), head)


def kernel(x):
    return pl.pallas_call(
        _copy_set_kernel,
        grid=(_GRID,),
        in_specs=[pl.BlockSpec((_BLOCK_ROWS, _COLS), lambda i: (i, 0))],
        out_specs=pl.BlockSpec((_BLOCK_ROWS, _COLS), lambda i: (i, 0)),
        out_shape=jax.ShapeDtypeStruct((_ROWS, _COLS), x.dtype),
    )(x)
